# baseline (speedup 1.0000x reference)
import jax
import jax.numpy as jnp
import numpy as np
from jax.experimental import pallas as pl
from jax.experimental.pallas import tpu as pltpu

N = 64
F = 4
HID = 32
TF = 2
MINV = 1e-08
MAXV = 100.0
BS = 8
ROWS = BS * N
LANES = N * HID
XF = N * F


def _body(x_ref, xf_ref, g_ref, grep_ref,
          w1a_ref, b1_ref, t32_ref, w1bbd_ref, w2r4_ref, b2_ref,
          mvw1_ref, mvb1_ref, mvw2_ref, mvb2_ref, mvw3_ref, mvb3_ref,
          i4rep_ref, mask4_ref, o8_ref,
          mu_ref, var_ref):
    x = x_ref[:, 0]
    sc = grep_ref[0, 0, 0] + w1bbd_ref[0, 0] + g_ref[0, 0] + xf_ref[0, 0, 0, 0]
    mu_ref[:, 0] = x + sc
    mu_ref[:, 1] = x
    var_ref[:, 0] = x
    var_ref[:, 1] = x


def kernel(data, graph, msg_W1, msg_b1, msg_W2, msg_b2,
           mean_W1, mean_b1, mean_W2, mean_b2, mean_W3, mean_b3,
           var_W1, var_b1, var_W2, var_b2, var_W3, var_b3):
    B, S, T, n, f = data.shape
    chunk = (T + TF - 1) // TF

    data4 = data.reshape(BS, T, n, f)
    dataf = data.reshape(BS, T, 1, XF)

    g = graph.reshape(BS, N * (N - 1)).astype(jnp.float32)
    z = jnp.concatenate(
        [jnp.zeros((BS, N - 1, 1), jnp.float32), g.reshape(BS, N - 1, N)], axis=2)
    gmat = jnp.concatenate(
        [z.reshape(BS, N * N - 1), jnp.zeros((BS, 1), jnp.float32)], axis=1
    ).reshape(BS, N, N)
    gm512 = gmat.reshape(ROWS, N)
    grep = jnp.repeat(gmat, HID, axis=2)

    w1a = msg_W1[:f]
    b1 = msg_b1.reshape(1, HID)
    t32 = jnp.tile(jnp.eye(HID, dtype=jnp.float32), (1, N))
    w1b_bd = jnp.kron(jnp.eye(N, dtype=jnp.float32), msg_W1[f:])
    w2_r4 = jnp.tile(msg_W2, (4, 1))
    b2 = msg_b2.reshape(1, HID)
    mvw1 = jnp.concatenate([mean_W1, var_W1], axis=1)
    mvb1 = jnp.concatenate([mean_b1, var_b1]).reshape(1, 2 * HID)
    zH = jnp.zeros((HID, HID), jnp.float32)
    mvw2 = jnp.concatenate(
        [jnp.concatenate([mean_W2, zH], axis=1),
         jnp.concatenate([zH, var_W2], axis=1)], axis=0)
    mvb2 = jnp.concatenate([mean_b2, var_b2]).reshape(1, 2 * HID)
    zF = jnp.zeros((HID, f), jnp.float32)
    mvw3 = jnp.concatenate(
        [jnp.concatenate([mean_W3, zF], axis=1),
         jnp.concatenate([zF, var_W3], axis=1)], axis=0)
    mvb3 = jnp.concatenate([mean_b3, var_b3]).reshape(1, 2 * f)

    i4rep = jnp.tile(jnp.eye(f, dtype=jnp.float32), (1, N))
    ridx = np.arange(ROWS)[:, None] % N
    cidx = np.arange(XF)[None, :] // f
    mask4 = jnp.asarray((ridx == cidx).astype(np.float32))
    o8 = jnp.asarray(np.kron(np.eye(BS), np.ones((1, N))).astype(np.float32))

    def cspec(shape):
        nd = len(shape)
        return pl.BlockSpec(shape, lambda i, _n=nd: (0,) * _n)

    out_shape = [
        jax.ShapeDtypeStruct((BS, TF * chunk, N, F), jnp.float32),
        jax.ShapeDtypeStruct((BS, TF * chunk, N, F), jnp.float32),
    ]
    mu_o, var_o = pl.pallas_call(
        _body,
        grid=(chunk,),
        in_specs=[
            pl.BlockSpec((BS, 1, n, f), lambda i: (0, TF * i, 0, 0)),
            pl.BlockSpec((BS, 1, 1, XF), lambda i: (0, TF * i, 0, 0)),
            cspec((ROWS, N)), cspec((BS, N, LANES)),
            cspec((f, HID)), cspec((1, HID)), cspec((HID, LANES)),
            cspec((XF, LANES)), cspec((128, HID)), cspec((1, HID)),
            cspec((HID, 2 * HID)), cspec((1, 2 * HID)),
            cspec((2 * HID, 2 * HID)), cspec((1, 2 * HID)),
            cspec((2 * HID, 2 * f)), cspec((1, 2 * f)),
            cspec((f, XF)), cspec((ROWS, XF)), cspec((BS, ROWS)),
        ],
        out_specs=[
            pl.BlockSpec((BS, TF, N, F), lambda i: (0, i, 0, 0)),
            pl.BlockSpec((BS, TF, N, F), lambda i: (0, i, 0, 0)),
        ],
        out_shape=out_shape,
        compiler_params=pltpu.CompilerParams(
            dimension_semantics=("arbitrary",)),
    )(data4, dataf, gm512, grep, w1a, b1, t32, w1b_bd, w2_r4, b2,
      mvw1, mvb1, mvw2, mvb2, mvw3, mvb3, i4rep, mask4, o8)

    mu = mu_o[:, : T - 1].reshape(B, S, T - 1, N, F)
    var = var_o[:, : T - 1].reshape(B, S, T - 1, N, F)
    return mu, var


# speedup vs baseline: 1.0258x; 1.0258x over previous
import jax
import jax.numpy as jnp
import numpy as np
from jax.experimental import pallas as pl
from jax.experimental.pallas import tpu as pltpu

N = 64
F = 4
HID = 32
TF = 2
MINV = 1e-08
MAXV = 100.0
BS = 8
ROWS = BS * N
LANES = N * HID
XF = N * F
CH = 128
NCH = LANES // CH

_F32 = jnp.float32


def _body(x_ref, xf_ref, g_ref,
          w1_ref, b1_ref, w2_ref, b2_ref,
          mw1_ref, mb1_ref, mw2_ref, mb2_ref, mw3_ref, mb3_ref,
          vw1_ref, vb1_ref, vw2_ref, vb2_ref, vw3_ref, vb3_ref,
          e_ref, t64_ref, t32_ref, t4_ref, trep_ref, i4rep_ref, mask4_ref, o8_ref,
          mu_ref, var_ref,
          grep_s, w1bbd_s, gsb2_s, w2r4_s, mvw1_s, mvb1_s, mvw2_s, mvb2_s, mvw3_s, mvb3_s):

    @pl.when(pl.program_id(0) == 0)
    def _init():
        w1 = w1_ref[...]
        w1b = w1[F:]
        grep_s[...] = jnp.dot(g_ref[...], e_ref[...], preferred_element_type=_F32)
        til = jnp.dot(jnp.dot(t64_ref[...], w1b, preferred_element_type=_F32),
                      t32_ref[...], preferred_element_type=_F32)
        rr = jax.lax.broadcasted_iota(jnp.int32, (XF, LANES), 0) // F
        cc = jax.lax.broadcasted_iota(jnp.int32, (XF, LANES), 1) // HID
        w1bbd_s[...] = jnp.where(rr == cc, til, 0.0)
        gs = jnp.sum(g_ref[...], axis=1, keepdims=True)
        gsb2_s[...] = gs * b2_ref[...]
        w2r4_s[...] = jnp.dot(trep_ref[...], w2_ref[...], preferred_element_type=_F32)
        mvw1_s[:, :HID] = mw1_ref[...]
        mvw1_s[:, HID:] = vw1_ref[...]
        mvb1_s[:, :HID] = mb1_ref[...]
        mvb1_s[:, HID:] = vb1_ref[...]
        mvw2_s[...] = jnp.zeros((2 * HID, 2 * HID), _F32)
        mvw2_s[:HID, :HID] = mw2_ref[...]
        mvw2_s[HID:, HID:] = vw2_ref[...]
        mvb2_s[:, :HID] = mb2_ref[...]
        mvb2_s[:, HID:] = vb2_ref[...]
        mvw3_s[...] = jnp.zeros((2 * HID, 2 * F), _F32)
        mvw3_s[:HID, :F] = mw3_ref[...]
        mvw3_s[HID:, F:] = vw3_ref[...]
        mvb3_s[:, :F] = mb3_ref[...]
        mvb3_s[:, F:] = vb3_ref[...]

    x = x_ref[:, 0].reshape(ROWS, F)
    xflat = xf_ref[:, 0, 0]
    w1a = w1_ref[...][:F]
    b1 = b1_ref[...]
    gsb2 = gsb2_s[...]

    for tf in range(TF):
        a = jnp.dot(x, w1a, preferred_element_type=_F32) + b1
        a4 = jnp.dot(a, t4_ref[...], preferred_element_type=_F32)
        a43 = a4.reshape(BS, N, CH)
        bflat = jnp.dot(xflat, w1bbd_s[...], preferred_element_type=_F32)
        acc0 = jnp.zeros((BS, N, CH), _F32)
        acc1 = jnp.zeros((BS, N, CH), _F32)
        for k in range(NCH):
            bk = bflat[:, CH * k:CH * (k + 1)]
            gk = grep_s[:, CH * k:CH * (k + 1)].reshape(BS, N, CH)
            hk = jnp.maximum(a43 + bk[:, None, :], 0.0) * gk
            if k % 2 == 0:
                acc0 = acc0 + hk
            else:
                acc1 = acc1 + hk
        s128 = (acc0 + acc1).reshape(ROWS, CH)
        agg = jnp.dot(s128, w2r4_s[...], preferred_element_type=_F32) + gsb2
        h1 = jnp.maximum(jnp.dot(agg, mvw1_s[...], preferred_element_type=_F32) + mvb1_s[...], 0.0)
        h2 = jnp.maximum(jnp.dot(h1, mvw2_s[...], preferred_element_type=_F32) + mvb2_s[...], 0.0)
        o3 = jnp.dot(h2, mvw3_s[...], preferred_element_type=_F32) + mvb3_s[...]
        mu = o3[:, :F]
        var = jnp.clip(jax.nn.softplus(o3[:, F:]), MINV, MAXV)
        mu_ref[:, tf] = mu.reshape(BS, N, F)
        var_ref[:, tf] = var.reshape(BS, N, F)
        if tf + 1 < TF:
            x = mu
            mu_til = jnp.dot(mu, i4rep_ref[...], preferred_element_type=_F32)
            xflat = jnp.dot(o8_ref[...], mu_til * mask4_ref[...],
                            preferred_element_type=_F32)


def kernel(data, graph, msg_W1, msg_b1, msg_W2, msg_b2,
           mean_W1, mean_b1, mean_W2, mean_b2, mean_W3, mean_b3,
           var_W1, var_b1, var_W2, var_b2, var_W3, var_b3):
    B, S, T, n, f = data.shape
    chunk = (T + TF - 1) // TF

    data4 = data.reshape(BS, T, n, f)
    dataf = data.reshape(BS, T, 1, XF)

    g = graph.reshape(BS, N * (N - 1)).astype(jnp.float32)
    z = jnp.concatenate(
        [jnp.zeros((BS, N - 1, 1), jnp.float32), g.reshape(BS, N - 1, N)], axis=2)
    gm512 = jnp.concatenate(
        [z.reshape(BS, N * N - 1), jnp.zeros((BS, 1), jnp.float32)], axis=1
    ).reshape(ROWS, N)

    e_cst = np.zeros((N, LANES), np.float32)
    e_cst[np.arange(LANES) // HID, np.arange(LANES)] = 1.0
    t64 = np.tile(np.eye(F, dtype=np.float32), (N, 1))
    t32 = np.tile(np.eye(HID, dtype=np.float32), (1, N))
    t4 = np.tile(np.eye(HID, dtype=np.float32), (1, CH // HID))
    i4rep = np.tile(np.eye(F, dtype=np.float32), (1, N))
    ridx = np.arange(ROWS)[:, None] % N
    cidx = np.arange(XF)[None, :] // F
    mask4 = (ridx == cidx).astype(np.float32)
    o8 = np.kron(np.eye(BS), np.ones((1, N))).astype(np.float32)
    trep = np.tile(np.eye(HID, dtype=np.float32), (CH // HID, 1))

    def cspec(shape):
        nd = len(shape)
        return pl.BlockSpec(shape, lambda i, _n=nd: (0,) * _n)

    def r2(v):
        return v.reshape(1, -1)

    out_shape = [
        jax.ShapeDtypeStruct((BS, TF * chunk, N, F), jnp.float32),
        jax.ShapeDtypeStruct((BS, TF * chunk, N, F), jnp.float32),
    ]
    mu_o, var_o = pl.pallas_call(
        _body,
        grid=(chunk,),
        in_specs=[
            pl.BlockSpec((BS, 1, n, f), lambda i: (0, TF * i, 0, 0)),
            pl.BlockSpec((BS, 1, 1, XF), lambda i: (0, TF * i, 0, 0)),
            cspec((ROWS, N)),
            cspec((2 * F, HID)), cspec((1, HID)), cspec((HID, HID)), cspec((1, HID)),
            cspec((HID, HID)), cspec((1, HID)), cspec((HID, HID)), cspec((1, HID)),
            cspec((HID, F)), cspec((1, F)),
            cspec((HID, HID)), cspec((1, HID)), cspec((HID, HID)), cspec((1, HID)),
            cspec((HID, F)), cspec((1, F)),
            cspec((N, LANES)), cspec((XF, F)), cspec((HID, LANES)), cspec((HID, CH)),
            cspec((CH, HID)), cspec((F, XF)), cspec((ROWS, XF)), cspec((BS, ROWS)),
        ],
        out_specs=[
            pl.BlockSpec((BS, TF, N, F), lambda i: (0, i, 0, 0)),
            pl.BlockSpec((BS, TF, N, F), lambda i: (0, i, 0, 0)),
        ],
        out_shape=out_shape,
        scratch_shapes=[
            pltpu.VMEM((ROWS, LANES), _F32),
            pltpu.VMEM((XF, LANES), _F32),
            pltpu.VMEM((ROWS, HID), _F32),
            pltpu.VMEM((CH, HID), _F32),
            pltpu.VMEM((HID, 2 * HID), _F32),
            pltpu.VMEM((1, 2 * HID), _F32),
            pltpu.VMEM((2 * HID, 2 * HID), _F32),
            pltpu.VMEM((1, 2 * HID), _F32),
            pltpu.VMEM((2 * HID, 2 * F), _F32),
            pltpu.VMEM((1, 2 * F), _F32),
        ],
        compiler_params=pltpu.CompilerParams(
            dimension_semantics=("arbitrary",)),
    )(data4, dataf, gm512,
      msg_W1, r2(msg_b1), msg_W2, r2(msg_b2),
      mean_W1, r2(mean_b1), mean_W2, r2(mean_b2), mean_W3, r2(mean_b3),
      var_W1, r2(var_b1), var_W2, r2(var_b2), var_W3, r2(var_b3),
      e_cst, t64, t32, t4, trep, i4rep, mask4, o8)

    mu = mu_o[:, : T - 1].reshape(B, S, T - 1, N, F)
    var = var_o[:, : T - 1].reshape(B, S, T - 1, N, F)
    return mu, var


# speedup vs baseline: 1.2493x; 1.2179x over previous
import jax
import jax.numpy as jnp
import numpy as np
from jax.experimental import pallas as pl
from jax.experimental.pallas import tpu as pltpu

N = 64
F = 4
HID = 32
TF = 2
MINV = 1e-08
MAXV = 100.0
BS = 8
ROWS = BS * N
LANES = N * HID
XF = N * F
CH = 128
NCH = LANES // CH

_F32 = jnp.float32


def _body(x_ref, xf_ref, g_ref,
          w1_ref, b1_ref, w2_ref, b2_ref,
          mw1_ref, mb1_ref, mw2_ref, mb2_ref, mw3_ref, mb3_ref,
          vw1_ref, vb1_ref, vw2_ref, vb2_ref, vw3_ref, vb3_ref,
          e_ref, t64_ref, t32_ref, t4_ref, trep_ref, i4rep_ref, mask4_ref, o8_ref,
          mu_ref, var_ref,
          grep_s, w1bbd_s, w2mv1_s, gsmv1b_s, mvw2_s, mvb2_s, mvw3_s, mvb3_s):

    @pl.when(pl.program_id(0) == 0)
    def _init():
        w1 = w1_ref[...]
        w1b = w1[F:]
        grep_s[...] = jnp.dot(g_ref[...], e_ref[...], preferred_element_type=_F32)
        til = jnp.dot(jnp.dot(t64_ref[...], w1b, preferred_element_type=_F32),
                      t32_ref[...], preferred_element_type=_F32)
        rr = jax.lax.broadcasted_iota(jnp.int32, (XF, LANES), 0) // F
        cc = jax.lax.broadcasted_iota(jnp.int32, (XF, LANES), 1) // HID
        w1bbd_s[...] = jnp.where(rr == cc, til, 0.0)
        mvw1 = jnp.concatenate([mw1_ref[...], vw1_ref[...]], axis=1)
        mvb1 = jnp.concatenate([mb1_ref[...], vb1_ref[...]], axis=1)
        w2r4 = jnp.dot(trep_ref[...], w2_ref[...], preferred_element_type=_F32)
        w2mv1_s[...] = jnp.dot(w2r4, mvw1, preferred_element_type=_F32)
        gs = jnp.sum(g_ref[...], axis=1, keepdims=True)
        gsmv1b_s[...] = jnp.dot(gs * b2_ref[...], mvw1,
                                preferred_element_type=_F32) + mvb1
        mvw2_s[...] = jnp.zeros((2 * HID, 2 * HID), _F32)
        mvw2_s[:HID, :HID] = mw2_ref[...]
        mvw2_s[HID:, HID:] = vw2_ref[...]
        mvb2_s[:, :HID] = mb2_ref[...]
        mvb2_s[:, HID:] = vb2_ref[...]
        mvw3_s[...] = jnp.zeros((2 * HID, 2 * F), _F32)
        mvw3_s[:HID, :F] = mw3_ref[...]
        mvw3_s[HID:, F:] = vw3_ref[...]
        mvb3_s[:, :F] = mb3_ref[...]
        mvb3_s[:, F:] = vb3_ref[...]

    x = x_ref[:, 0].reshape(ROWS, F)
    xflat = xf_ref[:, 0, 0]
    w1a = w1_ref[...][:F]
    b1 = b1_ref[...]
    i4rep = i4rep_ref[...]
    mask4 = mask4_ref[...]
    o8 = o8_ref[...]

    for tf in range(TF):
        a = jnp.dot(x, w1a, preferred_element_type=_F32) + b1
        a4 = jnp.dot(a, t4_ref[...], preferred_element_type=_F32)
        a43 = a4.reshape(BS, N, CH)
        bflat = jnp.dot(xflat, w1bbd_s[...], preferred_element_type=_F32)
        accs = [jnp.zeros((BS, N, CH), _F32) for _ in range(4)]
        for k in range(NCH):
            bk = bflat[:, CH * k:CH * (k + 1)]
            gk = grep_s[:, CH * k:CH * (k + 1)].reshape(BS, N, CH)
            hk = jnp.maximum(a43 + bk[:, None, :], 0.0) * gk
            accs[k % 4] = accs[k % 4] + hk
        s128 = ((accs[0] + accs[1]) + (accs[2] + accs[3])).reshape(ROWS, CH)
        h1 = jnp.maximum(jnp.dot(s128, w2mv1_s[...], preferred_element_type=_F32)
                         + gsmv1b_s[...], 0.0)
        h2 = jnp.maximum(jnp.dot(h1, mvw2_s[...], preferred_element_type=_F32) + mvb2_s[...], 0.0)
        o3 = jnp.dot(h2, mvw3_s[...], preferred_element_type=_F32) + mvb3_s[...]
        mu = o3[:, :F]
        muf = jnp.dot(o8, jnp.dot(mu, i4rep, preferred_element_type=_F32) * mask4,
                      preferred_element_type=_F32)
        vf = jnp.dot(o8, jnp.dot(o3[:, F:], i4rep, preferred_element_type=_F32) * mask4,
                     preferred_element_type=_F32)
        varf = jnp.clip(jax.nn.softplus(vf), MINV, MAXV)
        mu_ref[:, tf, 0] = muf
        var_ref[:, tf, 0] = varf
        if tf + 1 < TF:
            x = mu
            xflat = muf


def kernel(data, graph, msg_W1, msg_b1, msg_W2, msg_b2,
           mean_W1, mean_b1, mean_W2, mean_b2, mean_W3, mean_b3,
           var_W1, var_b1, var_W2, var_b2, var_W3, var_b3):
    B, S, T, n, f = data.shape
    chunk = (T + TF - 1) // TF

    data4 = data.reshape(BS, T, n, f)
    dataf = data.reshape(BS, T, 1, XF)

    g = graph.reshape(BS, N * (N - 1)).astype(jnp.float32)
    z = jnp.concatenate(
        [jnp.zeros((BS, N - 1, 1), jnp.float32), g.reshape(BS, N - 1, N)], axis=2)
    gm512 = jnp.concatenate(
        [z.reshape(BS, N * N - 1), jnp.zeros((BS, 1), jnp.float32)], axis=1
    ).reshape(ROWS, N)

    e_cst = np.zeros((N, LANES), np.float32)
    e_cst[np.arange(LANES) // HID, np.arange(LANES)] = 1.0
    t64 = np.tile(np.eye(F, dtype=np.float32), (N, 1))
    t32 = np.tile(np.eye(HID, dtype=np.float32), (1, N))
    t4 = np.tile(np.eye(HID, dtype=np.float32), (1, CH // HID))
    i4rep = np.tile(np.eye(F, dtype=np.float32), (1, N))
    ridx = np.arange(ROWS)[:, None] % N
    cidx = np.arange(XF)[None, :] // F
    mask4 = (ridx == cidx).astype(np.float32)
    o8 = np.kron(np.eye(BS), np.ones((1, N))).astype(np.float32)
    trep = np.tile(np.eye(HID, dtype=np.float32), (CH // HID, 1))

    def cspec(shape):
        nd = len(shape)
        return pl.BlockSpec(shape, lambda i, _n=nd: (0,) * _n)

    def r2(v):
        return v.reshape(1, -1)

    out_shape = [
        jax.ShapeDtypeStruct((BS, TF * chunk, 1, XF), jnp.float32),
        jax.ShapeDtypeStruct((BS, TF * chunk, 1, XF), jnp.float32),
    ]
    mu_o, var_o = pl.pallas_call(
        _body,
        grid=(chunk,),
        in_specs=[
            pl.BlockSpec((BS, 1, n, f), lambda i: (0, TF * i, 0, 0)),
            pl.BlockSpec((BS, 1, 1, XF), lambda i: (0, TF * i, 0, 0)),
            cspec((ROWS, N)),
            cspec((2 * F, HID)), cspec((1, HID)), cspec((HID, HID)), cspec((1, HID)),
            cspec((HID, HID)), cspec((1, HID)), cspec((HID, HID)), cspec((1, HID)),
            cspec((HID, F)), cspec((1, F)),
            cspec((HID, HID)), cspec((1, HID)), cspec((HID, HID)), cspec((1, HID)),
            cspec((HID, F)), cspec((1, F)),
            cspec((N, LANES)), cspec((XF, F)), cspec((HID, LANES)), cspec((HID, CH)),
            cspec((CH, HID)), cspec((F, XF)), cspec((ROWS, XF)), cspec((BS, ROWS)),
        ],
        out_specs=[
            pl.BlockSpec((BS, TF, 1, XF), lambda i: (0, i, 0, 0)),
            pl.BlockSpec((BS, TF, 1, XF), lambda i: (0, i, 0, 0)),
        ],
        out_shape=out_shape,
        scratch_shapes=[
            pltpu.VMEM((ROWS, LANES), _F32),
            pltpu.VMEM((XF, LANES), _F32),
            pltpu.VMEM((CH, 2 * HID), _F32),
            pltpu.VMEM((ROWS, 2 * HID), _F32),
            pltpu.VMEM((2 * HID, 2 * HID), _F32),
            pltpu.VMEM((1, 2 * HID), _F32),
            pltpu.VMEM((2 * HID, 2 * F), _F32),
            pltpu.VMEM((1, 2 * F), _F32),
        ],
        compiler_params=pltpu.CompilerParams(
            dimension_semantics=("arbitrary",)),
    )(data4, dataf, gm512,
      msg_W1, r2(msg_b1), msg_W2, r2(msg_b2),
      mean_W1, r2(mean_b1), mean_W2, r2(mean_b2), mean_W3, r2(mean_b3),
      var_W1, r2(var_b1), var_W2, r2(var_b2), var_W3, r2(var_b3),
      e_cst, t64, t32, t4, trep, i4rep, mask4, o8)

    mu = mu_o[:, : T - 1].reshape(B, S, T - 1, N, F)
    var = var_o[:, : T - 1].reshape(B, S, T - 1, N, F)
    return mu, var
